# Initial kernel scaffold; baseline (speedup 1.0000x reference)
import jax
import jax.numpy as jnp
from jax.experimental import pallas as pl


def kernel(label_ids, prompt_embeds, W1, b1, W2, b2):
    raise NotImplementedError("write your pallas kernel here")


# speedup vs baseline: 4.0300x; 4.0300x over previous
import functools

import jax
import jax.numpy as jnp
from jax import lax
from jax.experimental import pallas as pl
from jax.experimental.pallas import tpu as pltpu
from jax.experimental.pallas import tpu_sc as plsc

NUM_CLASSES = 100000
TEXT_DIM = 2048
HID = 256
CNT = 5
B = 4096
TOTAL = B * CNT

_SC_INFO = plsc.get_sparse_core_info()
_NC = _SC_INFO.num_cores
_NS = _SC_INFO.num_subcores
_NW = _NC * _NS
_BPW = TOTAL // _NW
_K = 32
_NCHUNK = _BPW // _K


@functools.partial(
    pl.kernel,
    mesh=plsc.VectorSubcoreMesh(core_axis_name="c", subcore_axis_name="s"),
    out_type=jax.ShapeDtypeStruct((TOTAL, TEXT_DIM), jnp.float32),
    scratch_types=[
        pltpu.VMEM((_BPW,), jnp.int32),
        pltpu.VMEM((_K, TEXT_DIM), jnp.float32),
        pltpu.SemaphoreType.DMA,
    ],
)
def _sc_gather(table_hbm, idx_hbm, out_hbm, idx_v, rows_v, sem):
    wid = lax.axis_index("s") * _NC + lax.axis_index("c")
    base = wid * _BPW
    pltpu.sync_copy(idx_hbm.at[pl.ds(base, _BPW)], idx_v)

    def body(c, carry):
        off = c * _K
        pltpu.async_copy(
            table_hbm.at[idx_v.at[pl.ds(off, _K)]], rows_v, sem
        ).wait()
        pltpu.sync_copy(rows_v, out_hbm.at[pl.ds(base + off, _K)])
        return carry

    lax.fori_loop(0, _NCHUNK, body, 0)


_BM = 1024


def _mlp_body(e_ref, w1_ref, b1_ref, w2_ref, b2_ref, o_ref):
    h = jnp.dot(e_ref[...], w1_ref[...], preferred_element_type=jnp.float32)
    h = h + b1_ref[...]
    h = h * lax.logistic(h)
    o = jnp.dot(h, w2_ref[...], preferred_element_type=jnp.float32)
    o_ref[...] = o + b2_ref[...]


def _mlp(e, w1, b1, w2, b2):
    nblk = TOTAL // _BM
    return pl.pallas_call(
        _mlp_body,
        grid=(nblk,),
        in_specs=[
            pl.BlockSpec((_BM, TEXT_DIM), lambda i: (i, 0)),
            pl.BlockSpec((TEXT_DIM, HID), lambda i: (0, 0)),
            pl.BlockSpec((1, HID), lambda i: (0, 0)),
            pl.BlockSpec((HID, HID), lambda i: (0, 0)),
            pl.BlockSpec((1, HID), lambda i: (0, 0)),
        ],
        out_specs=pl.BlockSpec((_BM, HID), lambda i: (i, 0)),
        out_shape=jax.ShapeDtypeStruct((TOTAL, HID), jnp.float32),
    )(e, w1, b1, w2, b2)


def kernel(label_ids, prompt_embeds, W1, b1, W2, b2):
    ids = label_ids.reshape(-1).astype(jnp.int32)
    gathered = _sc_gather(prompt_embeds, ids)
    out = _mlp(gathered, W1, b1.reshape(1, HID), W2, b2.reshape(1, HID))
    return out.reshape(B, CNT * HID)


# speedup vs baseline: 4.2501x; 1.0546x over previous
import functools

import jax
import jax.numpy as jnp
from jax import lax
from jax.experimental import pallas as pl
from jax.experimental.pallas import tpu as pltpu
from jax.experimental.pallas import tpu_sc as plsc

NUM_CLASSES = 100000
TEXT_DIM = 2048
HID = 256
CNT = 5
B = 4096
TOTAL = B * CNT

_SC_INFO = plsc.get_sparse_core_info()
_NC = _SC_INFO.num_cores
_NS = _SC_INFO.num_subcores
_NW = _NC * _NS
_BPW = TOTAL // _NW
_K = 16
_NCHUNK = _BPW // _K
_NHALF = _NCHUNK // 2


@functools.partial(
    pl.kernel,
    mesh=plsc.VectorSubcoreMesh(core_axis_name="c", subcore_axis_name="s"),
    out_type=jax.ShapeDtypeStruct((TOTAL, TEXT_DIM), jnp.float32),
    scratch_types=[
        pltpu.VMEM((_BPW,), jnp.int32),
        pltpu.VMEM((_K, TEXT_DIM), jnp.float32),
        pltpu.VMEM((_K, TEXT_DIM), jnp.float32),
        pltpu.SemaphoreType.DMA,
        pltpu.SemaphoreType.DMA,
        pltpu.SemaphoreType.DMA,
        pltpu.SemaphoreType.DMA,
    ],
)
def _sc_gather(table_hbm, idx_hbm, out_hbm, idx_v, buf0, buf1,
               gsem0, gsem1, wsem0, wsem1):
    wid = lax.axis_index("s") * _NC + lax.axis_index("c")
    base = wid * _BPW
    pltpu.sync_copy(idx_hbm.at[pl.ds(base, _BPW)], idx_v)

    def _gather(c, buf, sem):
        pltpu.async_copy(table_hbm.at[idx_v.at[pl.ds(c * _K, _K)]], buf, sem)

    def _wait(buf, sem):
        pltpu.make_async_copy(buf, out_hbm.at[pl.ds(base, _K)], sem).wait()

    _gather(0, buf0, gsem0)

    def body(i, carry):
        c0 = 2 * i

        @pl.when(i > 0)
        def _():
            _wait(buf1, wsem1)

        _gather(c0 + 1, buf1, gsem1)
        pltpu.make_async_copy(
            table_hbm.at[idx_v.at[pl.ds(0, _K)]], buf0, gsem0
        ).wait()
        pltpu.async_copy(buf0, out_hbm.at[pl.ds(base + c0 * _K, _K)], wsem0)

        @pl.when(i < _NHALF - 1)
        def _():
            _wait(buf0, wsem0)
            _gather(c0 + 2, buf0, gsem0)

        pltpu.make_async_copy(
            table_hbm.at[idx_v.at[pl.ds(0, _K)]], buf1, gsem1
        ).wait()
        pltpu.async_copy(
            buf1, out_hbm.at[pl.ds(base + (c0 + 1) * _K, _K)], wsem1
        )
        return carry

    lax.fori_loop(0, _NHALF, body, 0)
    _wait(buf0, wsem0)
    _wait(buf1, wsem1)


_BM = 1024


def _mlp_body(e_ref, w1_ref, b1_ref, w2_ref, b2_ref, o_ref):
    h = jnp.dot(e_ref[...], w1_ref[...], preferred_element_type=jnp.float32)
    h = h + b1_ref[...]
    h = h * lax.logistic(h)
    o = jnp.dot(h, w2_ref[...], preferred_element_type=jnp.float32)
    o_ref[...] = o + b2_ref[...]


def _mlp(e, w1, b1, w2, b2):
    nblk = TOTAL // _BM
    return pl.pallas_call(
        _mlp_body,
        grid=(nblk,),
        in_specs=[
            pl.BlockSpec((_BM, TEXT_DIM), lambda i: (i, 0)),
            pl.BlockSpec((TEXT_DIM, HID), lambda i: (0, 0)),
            pl.BlockSpec((1, HID), lambda i: (0, 0)),
            pl.BlockSpec((HID, HID), lambda i: (0, 0)),
            pl.BlockSpec((1, HID), lambda i: (0, 0)),
        ],
        out_specs=pl.BlockSpec((_BM, HID), lambda i: (i, 0)),
        out_shape=jax.ShapeDtypeStruct((TOTAL, HID), jnp.float32),
    )(e, w1, b1, w2, b2)


def kernel(label_ids, prompt_embeds, W1, b1, W2, b2):
    ids = label_ids.reshape(-1).astype(jnp.int32)
    gathered = _sc_gather(prompt_embeds, ids)
    out = _mlp(gathered, W1, b1.reshape(1, HID), W2, b2.reshape(1, HID))
    return out.reshape(B, CNT * HID)
